# Initial kernel scaffold; baseline (speedup 1.0000x reference)
import jax
import jax.numpy as jnp
from jax.experimental import pallas as pl


def kernel(x, edge_index, global_features, batch, W1, b1, W2, b2, W3, b3, Wp, bp, Wf, bf):
    raise NotImplementedError("write your pallas kernel here")


# speedup vs baseline: 19.9791x; 19.9791x over previous
import functools

import jax
import jax.numpy as jnp
from jax import lax
from jax.experimental import pallas as pl
from jax.experimental.pallas import tpu as pltpu
import jax.experimental.pallas.tpu_sc as plsc

_NC = 2
_NS = 16
_LANES = 16
_CHUNK = 128
_WCH = 4
_PAD_ROWS = 512
_ZROWS = 512

_BLK = 2000


def _zero_copies(rows):
  out, off = [], 0
  while off < rows:
    sz = min(_ZROWS, rows - off)
    out.append((off, sz))
    off += sz
  return out


def _acc_rows(half):
  want = half + _PAD_ROWS
  step = 8 * _NS
  return -(-want // step) * step


def _writeback(acc, out_hbm, half, row0, tile):
  slab = (-(-half // _NS) + 7) // 8 * 8
  last = half - (_NS - 1) * slab
  assert last > 0 and last % 8 == 0 and slab % 8 == 0

  @pl.when(tile < _NS - 1)
  def _():
    pltpu.sync_copy(acc.at[pl.ds(tile * slab, slab)],
                    out_hbm.at[pl.ds(row0 + tile * slab, slab)])

  @pl.when(tile == _NS - 1)
  def _():
    pltpu.sync_copy(acc.at[pl.ds((_NS - 1) * slab, last)],
                    out_hbm.at[pl.ds(row0 + (_NS - 1) * slab, last)])


@functools.cache
def _deg_kernel(n_nodes, n_chunks):
  half = n_nodes // 2
  rows_spmem = _acc_rows(half)
  chunks_per_tile = n_chunks // _NS
  windows = chunks_per_tile // _WCH
  mesh = plsc.VectorSubcoreMesh(core_axis_name="c", subcore_axis_name="s")

  @functools.partial(
      pl.kernel,
      out_type=jax.ShapeDtypeStruct((n_nodes, 8), jnp.float32),
      mesh=mesh,
      scratch_types=[
          pltpu.VMEM((_WCH, _CHUNK), jnp.int32),
          pltpu.VMEM((_CHUNK, 8), jnp.float32),
          pltpu.VMEM_SHARED((rows_spmem, 8), jnp.float32),
      ],
      compiler_params=pltpu.CompilerParams(use_tc_tiling_on_sc=False),
  )
  def k(dst_hbm, ones_hbm, zeros_hbm, out_hbm, idx_dst, ones_v, acc):
    core = lax.axis_index("c")
    tile = lax.axis_index("s")
    row0 = core * half

    pltpu.sync_copy(ones_hbm, ones_v)
    zoff = tile * (rows_spmem // _NS)
    for off, sz in _zero_copies(rows_spmem // _NS):
      pltpu.sync_copy(zeros_hbm.at[pl.ds(0, sz)], acc.at[pl.ds(zoff + off, sz)])
    plsc.subcore_barrier()

    def window(w, carry):
      base = tile * chunks_per_tile + w * _WCH
      pltpu.sync_copy(dst_hbm.at[pl.ds(base, _WCH)], idx_dst)
      for j in range(_WCH):
        for v in range(_CHUNK // _LANES):
          sl = pl.ds(v * _LANES, _LANES)
          d = idx_dst[j, sl]
          loc = d - row0
          oob = (loc < 0) | (loc >= half)
          pad = half + (d & (_PAD_ROWS - 1))
          idx_dst[j, sl] = jnp.where(oob, pad, loc)
      for j in range(_WCH):
        pltpu.sync_copy(ones_v, acc.at[idx_dst.at[j]], add=True)
      return carry

    lax.fori_loop(0, windows, window, 0)
    plsc.subcore_barrier()
    _writeback(acc, out_hbm, half, row0, tile)

  return k


@functools.cache
def _edge_scatter_kernel(n_nodes, n_chunks, feat):
  half = n_nodes // 2
  rows_spmem = _acc_rows(half)
  chunks_per_tile = n_chunks // _NS
  windows = chunks_per_tile // _WCH
  mesh = plsc.VectorSubcoreMesh(core_axis_name="c", subcore_axis_name="s")

  @functools.partial(
      pl.kernel,
      out_type=jax.ShapeDtypeStruct((n_nodes, feat), jnp.float32),
      mesh=mesh,
      scratch_types=[
          pltpu.VMEM((_WCH, _CHUNK), jnp.int32),
          pltpu.VMEM((_WCH, _CHUNK), jnp.int32),
          pltpu.VMEM((_WCH, _CHUNK, feat), jnp.float32),
          pltpu.VMEM_SHARED((rows_spmem, feat), jnp.float32),
          pltpu.SemaphoreType.DMA,
      ],
      compiler_params=pltpu.CompilerParams(use_tc_tiling_on_sc=False),
  )
  def k(src_hbm, dst_hbm, u_hbm, zeros_hbm, out_hbm,
        idx_src, idx_dst, rows, acc, sem):
    core = lax.axis_index("c")
    tile = lax.axis_index("s")
    row0 = core * half

    zoff = tile * (rows_spmem // _NS)
    for off, sz in _zero_copies(rows_spmem // _NS):
      pltpu.sync_copy(zeros_hbm.at[pl.ds(0, sz)], acc.at[pl.ds(zoff + off, sz)])
    plsc.subcore_barrier()

    def window(w, carry):
      base = tile * chunks_per_tile + w * _WCH
      pltpu.sync_copy(src_hbm.at[pl.ds(base, _WCH)], idx_src)
      pltpu.sync_copy(dst_hbm.at[pl.ds(base, _WCH)], idx_dst)
      descs = [
          pltpu.async_copy(u_hbm.at[idx_src.at[j]], rows.at[j], sem)
          for j in range(_WCH)
      ]
      for j in range(_WCH):
        for v in range(_CHUNK // _LANES):
          sl = pl.ds(v * _LANES, _LANES)
          d = idx_dst[j, sl]
          loc = d - row0
          oob = (loc < 0) | (loc >= half)
          pad = half + (d & (_PAD_ROWS - 1))
          idx_dst[j, sl] = jnp.where(oob, pad, loc)
      for j in range(_WCH):
        descs[j].wait()
        pltpu.sync_copy(rows.at[j], acc.at[idx_dst.at[j]], add=True)
      return carry

    lax.fori_loop(0, windows, window, 0)
    plsc.subcore_barrier()
    _writeback(acc, out_hbm, half, row0, tile)

  return k


def _tc_first_body(x_ref, deg_ref, w_ref, o_ref):
  dinv = lax.rsqrt(deg_ref[...] + 1.0)
  o_ref[...] = jnp.dot(x_ref[...], w_ref[...],
                       preferred_element_type=jnp.float32) * dinv


def _tc_mid_body(agg_ref, u_ref, deg_ref, b_ref, w_ref, o_ref):
  dinv = lax.rsqrt(deg_ref[...] + 1.0)
  h = jnp.maximum((agg_ref[...] + u_ref[...]) * dinv + b_ref[...], 0.0)
  o_ref[...] = jnp.dot(h, w_ref[...], preferred_element_type=jnp.float32) * dinv


def _tc_pool_body(agg_ref, u_ref, deg_ref, b_ref, o_ref):
  i = pl.program_id(0)
  dinv = lax.rsqrt(deg_ref[...] + 1.0)
  h = jnp.maximum((agg_ref[...] + u_ref[...]) * dinv + b_ref[...], 0.0)
  s = jnp.sum(h, axis=0, keepdims=True)

  @pl.when(i == 0)
  def _():
    o_ref[...] = s

  @pl.when(i > 0)
  def _():
    o_ref[...] = o_ref[...] + s


def _tc_head_body(inv_n, p_ref, gf_ref, wp_ref, bp_ref, wfa_ref, wfb_ref,
                  bf_ref, o_ref):
  ge = jnp.maximum(
      jnp.dot(p_ref[...] * inv_n, wp_ref[...],
              preferred_element_type=jnp.float32) + bp_ref[...], 0.0)
  o = (jnp.dot(ge, wfa_ref[...], preferred_element_type=jnp.float32) +
       jnp.dot(gf_ref[...], wfb_ref[...], preferred_element_type=jnp.float32) +
       bf_ref[...])
  o_ref[...] = jnp.maximum(o, 0.0)


def _row_spec(cols):
  return pl.BlockSpec((_BLK, cols), lambda i: (i, 0))


def _full_spec(shape):
  return pl.BlockSpec(shape, lambda i: tuple(0 for _ in shape))


def _tc_first(x, deg, w):
  n, d_in = x.shape
  h = w.shape[1]
  return pl.pallas_call(
      _tc_first_body,
      grid=(n // _BLK,),
      in_specs=[_row_spec(d_in), _row_spec(1), _full_spec(w.shape)],
      out_specs=_row_spec(h),
      out_shape=jax.ShapeDtypeStruct((n, h), jnp.float32),
  )(x, deg, w)


def _tc_mid(agg, u, deg, b, w):
  n, h = u.shape
  h2 = w.shape[1]
  return pl.pallas_call(
      _tc_mid_body,
      grid=(n // _BLK,),
      in_specs=[_row_spec(h), _row_spec(h), _row_spec(1),
                _full_spec(b.shape), _full_spec(w.shape)],
      out_specs=_row_spec(h2),
      out_shape=jax.ShapeDtypeStruct((n, h2), jnp.float32),
  )(agg, u, deg, b, w)


def _tc_pool(agg, u, deg, b):
  n, h = u.shape
  return pl.pallas_call(
      _tc_pool_body,
      grid=(n // _BLK,),
      in_specs=[_row_spec(h), _row_spec(h), _row_spec(1), _full_spec(b.shape)],
      out_specs=pl.BlockSpec((1, h), lambda i: (0, 0)),
      out_shape=jax.ShapeDtypeStruct((1, h), jnp.float32),
  )(agg, u, deg, b)


def _tc_head(pooled, gf, wp, bp, wfa, wfb, bf, n):
  out_d = bf.shape[1]
  return pl.pallas_call(
      functools.partial(_tc_head_body, 1.0 / n),
      in_specs=[pl.BlockSpec(a.shape, lambda: (0,) * a.ndim)
                for a in (pooled, gf, wp, bp, wfa, wfb, bf)],
      out_specs=pl.BlockSpec((1, out_d), lambda: (0, 0)),
      out_shape=jax.ShapeDtypeStruct((1, out_d), jnp.float32),
  )(pooled, gf, wp, bp, wfa, wfb, bf)


def kernel(x, edge_index, global_features, batch,
           W1, b1, W2, b2, W3, b3, Wp, bp, Wf, bf):
  n = x.shape[0]
  e = edge_index.shape[1]
  src, dst = edge_index[0], edge_index[1]

  group = _NS * _WCH * _CHUNK
  e_pad = (-e) % group
  if e_pad:
    ar = jnp.arange(e_pad, dtype=jnp.int32)
    src = jnp.concatenate([src, ar % n])
    dst = jnp.concatenate([dst, n + ar])
  srcm = src.reshape(-1, _CHUNK)
  dstm = dst.reshape(-1, _CHUNK)
  n_chunks = srcm.shape[0]

  ones8 = jnp.ones((_CHUNK, 8), jnp.float32)
  zeros8 = jnp.zeros((_ZROWS, 8), jnp.float32)
  zeros32 = jnp.zeros((_ZROWS, W1.shape[1]), jnp.float32)

  deg8 = _deg_kernel(n, n_chunks)(dstm, ones8, zeros8)
  deg = deg8[:, :1]

  scat = _edge_scatter_kernel(n, n_chunks, W1.shape[1])

  u = _tc_first(x, deg, W1)
  agg = scat(srcm, dstm, u, zeros32)
  u = _tc_mid(agg, u, deg, b1.reshape(1, -1), W2)
  agg = scat(srcm, dstm, u, zeros32)
  u = _tc_mid(agg, u, deg, b2.reshape(1, -1), W3)
  agg = scat(srcm, dstm, u, zeros32)
  pooled = _tc_pool(agg, u, deg, b3.reshape(1, -1))

  g = Wp.shape[1]
  out = _tc_head(pooled, global_features, Wp, bp.reshape(1, -1),
                 Wf[:g], Wf[g:], bf.reshape(1, -1), float(n))
  return out


# speedup vs baseline: 22.0483x; 1.1036x over previous
import functools

import jax
import jax.numpy as jnp
from jax import lax
from jax.experimental import pallas as pl
from jax.experimental.pallas import tpu as pltpu
import jax.experimental.pallas.tpu_sc as plsc

_NC = 2
_NS = 16
_LANES = 16
_CHUNK = 128
_WCH = 3
_PAD_ROWS = 512
_ZROWS = 512

_BLK = 2000


def _zero_copies(rows):
  out, off = [], 0
  while off < rows:
    sz = min(_ZROWS, rows - off)
    out.append((off, sz))
    off += sz
  return out


def _acc_rows(half):
  want = half + _PAD_ROWS
  step = 8 * _NS
  return -(-want // step) * step


def _writeback(acc, out_hbm, half, row0, tile):
  slab = (-(-half // _NS) + 7) // 8 * 8
  last = half - (_NS - 1) * slab
  assert last > 0 and last % 8 == 0 and slab % 8 == 0

  @pl.when(tile < _NS - 1)
  def _():
    pltpu.sync_copy(acc.at[pl.ds(tile * slab, slab)],
                    out_hbm.at[pl.ds(row0 + tile * slab, slab)])

  @pl.when(tile == _NS - 1)
  def _():
    pltpu.sync_copy(acc.at[pl.ds((_NS - 1) * slab, last)],
                    out_hbm.at[pl.ds(row0 + (_NS - 1) * slab, last)])


@functools.cache
def _deg_kernel(n_nodes, n_chunks):
  half = n_nodes // 2
  rows_spmem = _acc_rows(half)
  chunks_per_tile = n_chunks // _NS
  windows = chunks_per_tile // _WCH
  mesh = plsc.VectorSubcoreMesh(core_axis_name="c", subcore_axis_name="s")

  @functools.partial(
      pl.kernel,
      out_type=jax.ShapeDtypeStruct((n_nodes, 8), jnp.float32),
      mesh=mesh,
      scratch_types=[
          pltpu.VMEM((_WCH, _CHUNK), jnp.int32),
          pltpu.VMEM((_CHUNK, 8), jnp.float32),
          pltpu.VMEM_SHARED((rows_spmem, 8), jnp.float32),
      ],
      compiler_params=pltpu.CompilerParams(use_tc_tiling_on_sc=False),
  )
  def k(dst_hbm, ones_hbm, zeros_hbm, out_hbm, idx_dst, ones_v, acc):
    core = lax.axis_index("c")
    tile = lax.axis_index("s")
    row0 = core * half

    pltpu.sync_copy(ones_hbm, ones_v)
    zoff = tile * (rows_spmem // _NS)
    for off, sz in _zero_copies(rows_spmem // _NS):
      pltpu.sync_copy(zeros_hbm.at[pl.ds(0, sz)], acc.at[pl.ds(zoff + off, sz)])
    plsc.subcore_barrier()

    def window(w, carry):
      base = tile * chunks_per_tile + w * _WCH
      pltpu.sync_copy(dst_hbm.at[pl.ds(base, _WCH)], idx_dst)
      for j in range(_WCH):
        for v in range(_CHUNK // _LANES):
          sl = pl.ds(v * _LANES, _LANES)
          d = idx_dst[j, sl]
          loc = d - row0
          oob = (loc < 0) | (loc >= half)
          pad = half + (d & (_PAD_ROWS - 1))
          idx_dst[j, sl] = jnp.where(oob, pad, loc)
      for j in range(_WCH):
        pltpu.sync_copy(ones_v, acc.at[idx_dst.at[j]], add=True)
      return carry

    lax.fori_loop(0, windows, window, 0)
    plsc.subcore_barrier()
    _writeback(acc, out_hbm, half, row0, tile)

  return k


@functools.cache
def _edge_scatter_kernel(n_nodes, n_chunks, feat):
  half = n_nodes // 2
  rows_spmem = _acc_rows(half)
  chunks_per_tile = n_chunks // _NS
  windows = chunks_per_tile // _WCH
  assert chunks_per_tile % _WCH == 0 and windows % 2 == 1
  mesh = plsc.VectorSubcoreMesh(core_axis_name="c", subcore_axis_name="s")

  @functools.partial(
      pl.kernel,
      out_type=jax.ShapeDtypeStruct((n_nodes, feat), jnp.float32),
      mesh=mesh,
      scratch_types=[
          pltpu.VMEM((2, _WCH, _CHUNK), jnp.int32),
          pltpu.VMEM((2, _WCH, _CHUNK), jnp.int32),
          pltpu.VMEM((2, _WCH, _CHUNK, feat), jnp.float32),
          pltpu.VMEM_SHARED((rows_spmem, feat), jnp.float32),
          pltpu.SemaphoreType.DMA,
      ],
      compiler_params=pltpu.CompilerParams(use_tc_tiling_on_sc=False),
  )
  def k(src_hbm, dst_hbm, u_hbm, zeros_hbm, out_hbm,
        idx_src, idx_dst, rows, acc, sem):
    core = lax.axis_index("c")
    tile = lax.axis_index("s")
    row0 = core * half
    base0 = tile * chunks_per_tile

    zoff = tile * (rows_spmem // _NS)
    for off, sz in _zero_copies(rows_spmem // _NS):
      pltpu.sync_copy(zeros_hbm.at[pl.ds(0, sz)], acc.at[pl.ds(zoff + off, sz)])
    plsc.subcore_barrier()

    def load_localize(w, s):
      base = base0 + w * _WCH
      pltpu.sync_copy(src_hbm.at[pl.ds(base, _WCH)], idx_src.at[s])
      pltpu.sync_copy(dst_hbm.at[pl.ds(base, _WCH)], idx_dst.at[s])
      for j in range(_WCH):
        for v in range(_CHUNK // _LANES):
          sl = pl.ds(v * _LANES, _LANES)
          d = idx_dst[s, j, sl]
          loc = d - row0
          oob = (loc < 0) | (loc >= half)
          pad = half + (d & (_PAD_ROWS - 1))
          idx_dst[s, j, sl] = jnp.where(oob, pad, loc)

    def fire(s):
      for j in range(_WCH):
        pltpu.async_copy(u_hbm.at[idx_src.at[s, j]], rows.at[s, j], sem)

    def wait(s):
      for j in range(_WCH):
        pltpu.make_async_copy(u_hbm.at[idx_src.at[s, j]], rows.at[s, j],
                              sem).wait()

    def scat(s):
      for j in range(_WCH):
        pltpu.sync_copy(rows.at[s, j], acc.at[idx_dst.at[s, j]], add=True)

    load_localize(0, 0)
    fire(0)

    def double_body(i, carry):
      w = 2 * i
      load_localize(w + 1, 1)
      wait(0)
      fire(1)
      scat(0)
      load_localize(w + 2, 0)
      wait(1)
      fire(0)
      scat(1)
      return carry

    lax.fori_loop(0, (windows - 1) // 2, double_body, 0)
    wait(0)
    scat(0)

    plsc.subcore_barrier()
    _writeback(acc, out_hbm, half, row0, tile)

  return k


def _tc_first_body(x_ref, deg_ref, w_ref, o_ref):
  dinv = lax.rsqrt(deg_ref[...] + 1.0)
  o_ref[...] = jnp.dot(x_ref[...], w_ref[...],
                       preferred_element_type=jnp.float32) * dinv


def _tc_mid_body(agg_ref, u_ref, deg_ref, b_ref, w_ref, o_ref):
  dinv = lax.rsqrt(deg_ref[...] + 1.0)
  h = jnp.maximum((agg_ref[...] + u_ref[...]) * dinv + b_ref[...], 0.0)
  o_ref[...] = jnp.dot(h, w_ref[...], preferred_element_type=jnp.float32) * dinv


def _tc_pool_body(agg_ref, u_ref, deg_ref, b_ref, o_ref):
  i = pl.program_id(0)
  dinv = lax.rsqrt(deg_ref[...] + 1.0)
  h = jnp.maximum((agg_ref[...] + u_ref[...]) * dinv + b_ref[...], 0.0)
  s = jnp.sum(h, axis=0, keepdims=True)

  @pl.when(i == 0)
  def _():
    o_ref[...] = s

  @pl.when(i > 0)
  def _():
    o_ref[...] = o_ref[...] + s


def _tc_head_body(inv_n, p_ref, gf_ref, wp_ref, bp_ref, wfa_ref, wfb_ref,
                  bf_ref, o_ref):
  ge = jnp.maximum(
      jnp.dot(p_ref[...] * inv_n, wp_ref[...],
              preferred_element_type=jnp.float32) + bp_ref[...], 0.0)
  o = (jnp.dot(ge, wfa_ref[...], preferred_element_type=jnp.float32) +
       jnp.dot(gf_ref[...], wfb_ref[...], preferred_element_type=jnp.float32) +
       bf_ref[...])
  o_ref[...] = jnp.maximum(o, 0.0)


def _row_spec(cols):
  return pl.BlockSpec((_BLK, cols), lambda i: (i, 0))


def _full_spec(shape):
  return pl.BlockSpec(shape, lambda i: tuple(0 for _ in shape))


def _tc_first(x, deg, w):
  n, d_in = x.shape
  h = w.shape[1]
  return pl.pallas_call(
      _tc_first_body,
      grid=(n // _BLK,),
      in_specs=[_row_spec(d_in), _row_spec(1), _full_spec(w.shape)],
      out_specs=_row_spec(h),
      out_shape=jax.ShapeDtypeStruct((n, h), jnp.float32),
  )(x, deg, w)


def _tc_mid(agg, u, deg, b, w):
  n, h = u.shape
  h2 = w.shape[1]
  return pl.pallas_call(
      _tc_mid_body,
      grid=(n // _BLK,),
      in_specs=[_row_spec(h), _row_spec(h), _row_spec(1),
                _full_spec(b.shape), _full_spec(w.shape)],
      out_specs=_row_spec(h2),
      out_shape=jax.ShapeDtypeStruct((n, h2), jnp.float32),
  )(agg, u, deg, b, w)


def _tc_pool(agg, u, deg, b):
  n, h = u.shape
  return pl.pallas_call(
      _tc_pool_body,
      grid=(n // _BLK,),
      in_specs=[_row_spec(h), _row_spec(h), _row_spec(1), _full_spec(b.shape)],
      out_specs=pl.BlockSpec((1, h), lambda i: (0, 0)),
      out_shape=jax.ShapeDtypeStruct((1, h), jnp.float32),
  )(agg, u, deg, b)


def _tc_head(pooled, gf, wp, bp, wfa, wfb, bf, n):
  out_d = bf.shape[1]
  return pl.pallas_call(
      functools.partial(_tc_head_body, 1.0 / n),
      in_specs=[pl.BlockSpec(a.shape, lambda: (0,) * a.ndim)
                for a in (pooled, gf, wp, bp, wfa, wfb, bf)],
      out_specs=pl.BlockSpec((1, out_d), lambda: (0, 0)),
      out_shape=jax.ShapeDtypeStruct((1, out_d), jnp.float32),
  )(pooled, gf, wp, bp, wfa, wfb, bf)


def kernel(x, edge_index, global_features, batch,
           W1, b1, W2, b2, W3, b3, Wp, bp, Wf, bf):
  n = x.shape[0]
  e = edge_index.shape[1]
  src, dst = edge_index[0], edge_index[1]

  group = _NS * _WCH * _CHUNK
  e_pad = (-e) % group
  if ((e + e_pad) // group) % 2 == 0:
    e_pad += group
  if e_pad:
    ar = jnp.arange(e_pad, dtype=jnp.int32)
    src = jnp.concatenate([src, ar % n])
    dst = jnp.concatenate([dst, n + ar])
  srcm = src.reshape(-1, _CHUNK)
  dstm = dst.reshape(-1, _CHUNK)
  n_chunks = srcm.shape[0]

  ones8 = jnp.ones((_CHUNK, 8), jnp.float32)
  zeros8 = jnp.zeros((_ZROWS, 8), jnp.float32)
  zeros32 = jnp.zeros((_ZROWS, W1.shape[1]), jnp.float32)

  deg8 = _deg_kernel(n, n_chunks)(dstm, ones8, zeros8)
  deg = deg8[:, :1]

  scat = _edge_scatter_kernel(n, n_chunks, W1.shape[1])

  u = _tc_first(x, deg, W1)
  agg = scat(srcm, dstm, u, zeros32)
  u = _tc_mid(agg, u, deg, b1.reshape(1, -1), W2)
  agg = scat(srcm, dstm, u, zeros32)
  u = _tc_mid(agg, u, deg, b2.reshape(1, -1), W3)
  agg = scat(srcm, dstm, u, zeros32)
  pooled = _tc_pool(agg, u, deg, b3.reshape(1, -1))

  g = Wp.shape[1]
  out = _tc_head(pooled, global_features, Wp, bp.reshape(1, -1),
                 Wf[:g], Wf[g:], bf.reshape(1, -1), float(n))
  return out


# speedup vs baseline: 27.3627x; 1.2410x over previous
import functools

import jax
import jax.numpy as jnp
from jax import lax
from jax.experimental import pallas as pl
from jax.experimental.pallas import tpu as pltpu
import jax.experimental.pallas.tpu_sc as plsc

_NC = 2
_NS = 16
_CHUNK = 128
_SB = 12
_RING = 4
_DWCH = 24
_PAD_ROWS = 512
_ZROWS = 512

_BLK = 2000


def _zero_copies(rows):
  out, off = [], 0
  while off < rows:
    sz = min(_ZROWS, rows - off)
    out.append((off, sz))
    off += sz
  return out


def _acc_rows(half):
  want = half + _PAD_ROWS
  step = 8 * _NS
  return -(-want // step) * step


def _writeback(acc, out_hbm, half, row0, tile):
  slab = (-(-half // _NS) + 7) // 8 * 8
  last = half - (_NS - 1) * slab
  assert last > 0 and last % 8 == 0 and slab % 8 == 0

  @pl.when(tile < _NS - 1)
  def _():
    pltpu.sync_copy(acc.at[pl.ds(tile * slab, slab)],
                    out_hbm.at[pl.ds(row0 + tile * slab, slab)])

  @pl.when(tile == _NS - 1)
  def _():
    pltpu.sync_copy(acc.at[pl.ds((_NS - 1) * slab, last)],
                    out_hbm.at[pl.ds(row0 + (_NS - 1) * slab, last)])


@functools.cache
def _deg_kernel(n_nodes, n_chunks):
  half = n_nodes // 2
  rows_spmem = _acc_rows(half)
  chunks_per_tile = n_chunks // _NS
  windows = chunks_per_tile // _DWCH
  assert chunks_per_tile % _DWCH == 0
  mesh = plsc.VectorSubcoreMesh(core_axis_name="c", subcore_axis_name="s")

  @functools.partial(
      pl.kernel,
      out_type=jax.ShapeDtypeStruct((n_nodes, 8), jnp.float32),
      mesh=mesh,
      scratch_types=[
          pltpu.VMEM((_DWCH, _CHUNK), jnp.int32),
          pltpu.VMEM((_CHUNK, 8), jnp.float32),
          pltpu.VMEM_SHARED((rows_spmem, 8), jnp.float32),
          pltpu.SemaphoreType.DMA,
      ],
      compiler_params=pltpu.CompilerParams(use_tc_tiling_on_sc=False),
  )
  def k(dst2_hbm, ones_hbm, zeros_hbm, out_hbm, idx_dst, ones_v, acc, sem):
    core = lax.axis_index("c")
    tile = lax.axis_index("s")
    base0 = core * (_NS * chunks_per_tile) + tile * chunks_per_tile

    pltpu.sync_copy(ones_hbm, ones_v)
    zoff = tile * (rows_spmem // _NS)
    for off, sz in _zero_copies(rows_spmem // _NS):
      pltpu.sync_copy(zeros_hbm.at[pl.ds(0, sz)], acc.at[pl.ds(zoff + off, sz)])
    plsc.subcore_barrier()

    def window(w, carry):
      pltpu.sync_copy(dst2_hbm.at[pl.ds(base0 + w * _DWCH, _DWCH)], idx_dst)
      for j in range(_DWCH):
        pltpu.async_copy(ones_v, acc.at[idx_dst.at[j]], sem, add=True)
      for j in range(_DWCH):
        pltpu.make_async_copy(ones_v, acc.at[idx_dst.at[j]], sem).wait()
      return carry

    lax.fori_loop(0, windows, window, 0)
    plsc.subcore_barrier()
    _writeback(acc, out_hbm, half, core * half, tile)

  return k


@functools.cache
def _edge_scatter_kernel(n_nodes, n_chunks, feat):
  half = n_nodes // 2
  rows_spmem = _acc_rows(half)
  chunks_per_tile = n_chunks // _NS
  n_sb = chunks_per_tile // _SB
  assert chunks_per_tile % _SB == 0 and n_sb % 2 == 0 and n_sb >= 4
  mesh = plsc.VectorSubcoreMesh(core_axis_name="c", subcore_axis_name="s")

  @functools.partial(
      pl.kernel,
      out_type=jax.ShapeDtypeStruct((n_nodes, feat), jnp.float32),
      mesh=mesh,
      scratch_types=[
          pltpu.VMEM((2, _SB, _CHUNK), jnp.int32),
          pltpu.VMEM((2, _SB, _CHUNK), jnp.int32),
          pltpu.VMEM((_RING, _CHUNK, feat), jnp.float32),
          pltpu.VMEM((8, _CHUNK), jnp.int32),
          pltpu.VMEM_SHARED((rows_spmem, feat), jnp.float32),
          pltpu.SemaphoreType.DMA,
          pltpu.SemaphoreType.DMA,
      ],
      compiler_params=pltpu.CompilerParams(use_tc_tiling_on_sc=False),
  )
  def k(src_hbm, dst2_hbm, u_hbm, zeros_hbm, out_hbm,
        idx_src, idx_dst, rows, pad_idx, acc, sem_g, sem_s):
    core = lax.axis_index("c")
    tile = lax.axis_index("s")
    row0 = core * half
    base0 = tile * chunks_per_tile

    zoff = tile * (rows_spmem // _NS)
    for off, sz in _zero_copies(rows_spmem // _NS):
      pltpu.sync_copy(zeros_hbm.at[pl.ds(0, sz)], acc.at[pl.ds(zoff + off, sz)])
    lanes = lax.iota(jnp.int32, 16)
    for r in range(2):
      for v in range(_CHUNK // 16):
        pad_idx[r, pl.ds(v * 16, 16)] = half + ((r * _CHUNK + v * 16 + lanes)
                                                & (_PAD_ROWS - 1))
    plsc.subcore_barrier()

    def load_sb(s, p):
      base = base0 + s * _SB
      pltpu.sync_copy(src_hbm.at[pl.ds(base, _SB)], idx_src.at[p])
      pltpu.sync_copy(dst2_hbm.at[pl.ds(core * (_NS * chunks_per_tile) + base,
                                        _SB)], idx_dst.at[p])

    def fire_g(p, j, slot):
      pltpu.async_copy(u_hbm.at[idx_src.at[p, j]], rows.at[slot], sem_g)

    def wait_g(p, j, slot):
      pltpu.make_async_copy(u_hbm.at[idx_src.at[p, j]], rows.at[slot],
                            sem_g).wait()

    def fire_s(p, j, slot):
      pltpu.async_copy(rows.at[slot], acc.at[idx_dst.at[p, j]], sem_s,
                       add=True)

    def wait_s(p, j, slot):
      pltpu.make_async_copy(rows.at[slot], acc.at[idx_dst.at[p, j]],
                            sem_s).wait()

    def emit_sb(s, p, last=False):
      for j in range(_SB):
        slot = j % _RING
        ahead = (j + 2) % _RING
        wait_s(p, j, ahead)
        if j == 2 and not last:
          load_sb(s + 1, 1 - p)
        if not (last and j >= _SB - 2):
          if j < _SB - 2:
            fire_g(p, j + 2, ahead)
          else:
            fire_g(1 - p, j + 2 - _SB, ahead)
        wait_g(p, j, slot)
        fire_s(p, j, slot)
      if last:
        wait_s(p, _SB - 2, (_SB - 2) % _RING)
        wait_s(p, _SB - 1, (_SB - 1) % _RING)

    load_sb(0, 0)
    fire_g(0, 0, 0)
    fire_g(0, 1, 1)
    pltpu.async_copy(rows.at[2], acc.at[pad_idx.at[0]], sem_s, add=True)
    pltpu.async_copy(rows.at[3], acc.at[pad_idx.at[1]], sem_s, add=True)

    emit_sb(0, 0)
    emit_sb(1, 1)

    def pair(i, carry):
      s = 2 + 2 * i
      emit_sb(s, 0)
      emit_sb(s + 1, 1)
      return carry

    lax.fori_loop(0, (n_sb - 4) // 2, pair, 0)
    emit_sb(n_sb - 2, 0)
    emit_sb(n_sb - 1, 1, last=True)

    plsc.subcore_barrier()
    _writeback(acc, out_hbm, half, row0, tile)

  return k


def _tc_prep_body(half, dst_ref, o_ref):
  j = pl.program_id(0)
  d = dst_ref[...]
  pad = half + (d & (_PAD_ROWS - 1))

  @pl.when(j == 0)
  def _():
    o_ref[...] = jnp.where(d < half, d, pad)

  @pl.when(j == 1)
  def _():
    o_ref[...] = jnp.where((d >= half) & (d < 2 * half), d - half, pad)


def _tc_first_body(x_ref, deg_ref, w_ref, o_ref):
  dinv = lax.rsqrt(deg_ref[...] + 1.0)
  o_ref[...] = jnp.dot(x_ref[...], w_ref[...],
                       preferred_element_type=jnp.float32) * dinv


def _tc_mid_body(agg_ref, u_ref, deg_ref, b_ref, w_ref, o_ref):
  dinv = lax.rsqrt(deg_ref[...] + 1.0)
  h = jnp.maximum((agg_ref[...] + u_ref[...]) * dinv + b_ref[...], 0.0)
  o_ref[...] = jnp.dot(h, w_ref[...], preferred_element_type=jnp.float32) * dinv


def _tc_pool_body(agg_ref, u_ref, deg_ref, b_ref, o_ref):
  i = pl.program_id(0)
  dinv = lax.rsqrt(deg_ref[...] + 1.0)
  h = jnp.maximum((agg_ref[...] + u_ref[...]) * dinv + b_ref[...], 0.0)
  s = jnp.sum(h, axis=0, keepdims=True)

  @pl.when(i == 0)
  def _():
    o_ref[...] = s

  @pl.when(i > 0)
  def _():
    o_ref[...] = o_ref[...] + s


def _tc_head_body(inv_n, p_ref, gf_ref, wp_ref, bp_ref, wfa_ref, wfb_ref,
                  bf_ref, o_ref):
  ge = jnp.maximum(
      jnp.dot(p_ref[...] * inv_n, wp_ref[...],
              preferred_element_type=jnp.float32) + bp_ref[...], 0.0)
  o = (jnp.dot(ge, wfa_ref[...], preferred_element_type=jnp.float32) +
       jnp.dot(gf_ref[...], wfb_ref[...], preferred_element_type=jnp.float32) +
       bf_ref[...])
  o_ref[...] = jnp.maximum(o, 0.0)


def _row_spec(cols):
  return pl.BlockSpec((_BLK, cols), lambda i: (i, 0))


def _full_spec(shape):
  return pl.BlockSpec(shape, lambda i: tuple(0 for _ in shape))


def _tc_prep(dstm, half):
  n_chunks = dstm.shape[0]
  bc = 96
  assert n_chunks % bc == 0
  nb = n_chunks // bc
  return pl.pallas_call(
      functools.partial(_tc_prep_body, half),
      grid=(2, nb),
      in_specs=[pl.BlockSpec((bc, _CHUNK), lambda j, i: (i, 0))],
      out_specs=pl.BlockSpec((bc, _CHUNK), lambda j, i: (j * nb + i, 0)),
      out_shape=jax.ShapeDtypeStruct((2 * n_chunks, _CHUNK), jnp.int32),
  )(dstm)


def _tc_first(x, deg, w):
  n, d_in = x.shape
  h = w.shape[1]
  return pl.pallas_call(
      _tc_first_body,
      grid=(n // _BLK,),
      in_specs=[_row_spec(d_in), _row_spec(1), _full_spec(w.shape)],
      out_specs=_row_spec(h),
      out_shape=jax.ShapeDtypeStruct((n, h), jnp.float32),
  )(x, deg, w)


def _tc_mid(agg, u, deg, b, w):
  n, h = u.shape
  h2 = w.shape[1]
  return pl.pallas_call(
      _tc_mid_body,
      grid=(n // _BLK,),
      in_specs=[_row_spec(h), _row_spec(h), _row_spec(1),
               _full_spec(b.shape), _full_spec(w.shape)],
      out_specs=_row_spec(h2),
      out_shape=jax.ShapeDtypeStruct((n, h2), jnp.float32),
  )(agg, u, deg, b, w)


def _tc_pool(agg, u, deg, b):
  n, h = u.shape
  return pl.pallas_call(
      _tc_pool_body,
      grid=(n // _BLK,),
      in_specs=[_row_spec(h), _row_spec(h), _row_spec(1),
               _full_spec(b.shape)],
      out_specs=pl.BlockSpec((1, h), lambda i: (0, 0)),
      out_shape=jax.ShapeDtypeStruct((1, h), jnp.float32),
  )(agg, u, deg, b)


def _tc_head(pooled, gf, wp, bp, wfa, wfb, bf, n):
  out_d = bf.shape[1]
  return pl.pallas_call(
      functools.partial(_tc_head_body, 1.0 / n),
      in_specs=[pl.BlockSpec(a.shape, lambda: (0,) * a.ndim)
                for a in (pooled, gf, wp, bp, wfa, wfb, bf)],
      out_specs=pl.BlockSpec((1, out_d), lambda: (0, 0)),
      out_shape=jax.ShapeDtypeStruct((1, out_d), jnp.float32),
  )(pooled, gf, wp, bp, wfa, wfb, bf)


def kernel(x, edge_index, global_features, batch,
           W1, b1, W2, b2, W3, b3, Wp, bp, Wf, bf):
  n = x.shape[0]
  e = edge_index.shape[1]
  src, dst = edge_index[0], edge_index[1]

  group = _NS * _SB * _CHUNK * 2
  e_pad = (-e) % group
  if e_pad:
    ar = jnp.arange(e_pad, dtype=jnp.int32)
    src_s = jnp.concatenate([src, ar % n])
    dst_s = jnp.concatenate([dst, jnp.full((e_pad,), n, jnp.int32) + ar])
  else:
    src_s, dst_s = src, dst
  srcm = src_s.reshape(-1, _CHUNK)
  dstm = dst_s.reshape(-1, _CHUNK)
  n_chunks = srcm.shape[0]

  ones8 = jnp.ones((_CHUNK, 8), jnp.float32)
  zeros8 = jnp.zeros((_ZROWS, 8), jnp.float32)
  zeros32 = jnp.zeros((_ZROWS, W1.shape[1]), jnp.float32)

  dst2 = _tc_prep(dstm, n // 2)
  deg8 = _deg_kernel(n, n_chunks)(dst2, ones8, zeros8)
  deg = deg8[:, :1]

  scat = _edge_scatter_kernel(n, n_chunks, W1.shape[1])

  u = _tc_first(x, deg, W1)
  agg = scat(srcm, dst2, u, zeros32)
  u = _tc_mid(agg, u, deg, b1.reshape(1, -1), W2)
  agg = scat(srcm, dst2, u, zeros32)
  u = _tc_mid(agg, u, deg, b2.reshape(1, -1), W3)
  agg = scat(srcm, dst2, u, zeros32)
  pooled = _tc_pool(agg, u, deg, b3.reshape(1, -1))

  g = Wp.shape[1]
  out = _tc_head(pooled, global_features, Wp, bp.reshape(1, -1),
                 Wf[:g], Wf[g:], bf.reshape(1, -1), float(n))

  return out
